# baseline (speedup 1.0000x reference)
import functools

import jax
import jax.numpy as jnp
import numpy as np
from jax import lax
from jax.experimental import pallas as pl
from jax.experimental.pallas import tpu as pltpu
from jax.experimental.pallas import tpu_sc as plsc

_N = 10000
_E = 320000
_D = 128
_FC_IN = 16
_FC_HID = 64
_NUM_NEIGHBORS = 32.0
_SILU_NORM = 1.679177

_NC = 2
_NS = 16
_NW = _NC * _NS
_EPW = _E // _NW
_C = 40
_NPH = 5
_CPP = 50
_RPT = 624
_ZC = 8


def _node_body(x_ref, w1_ref, nf_ref):
    nf_ref[...] = jnp.dot(x_ref[...], w1_ref[...], preferred_element_type=jnp.float32)


def _half_words(es, ea, w1_ref, w2a_ref, w2b_ref):
    h = jnp.dot(es, w1_ref[...], preferred_element_type=jnp.float32)
    h = (h * (1.0 / (1.0 + jnp.exp(-h)))) * _SILU_NORM
    hb = h.astype(jnp.bfloat16)
    wpa = jnp.dot(hb, w2a_ref[...], preferred_element_type=jnp.float32) * ea
    wpb = jnp.dot(hb, w2b_ref[...], preferred_element_type=jnp.float32) * ea
    ia = lax.bitcast_convert_type(wpa.astype(jnp.bfloat16), jnp.uint16).astype(jnp.int32)
    ib = lax.bitcast_convert_type(wpb.astype(jnp.bfloat16), jnp.uint16).astype(jnp.int32)
    return ia | (ib << 16)


def _fc_body(esa_ref, esb_ref, eaa_ref, eab_ref, w1_ref, w2a_ref, w2b_ref, wp_ref):
    ia = _half_words(esa_ref[...], eaa_ref[...], w1_ref, w2a_ref, w2b_ref)
    ib = _half_words(esb_ref[...], eab_ref[...], w1_ref, w2a_ref, w2b_ref)
    wp_ref[...] = jnp.concatenate([ia, ib], axis=1)


_COLS_A = np.empty(_D // 2, np.int32)
_COLS_B = np.empty(_D // 2, np.int32)
for _m in range(_D // 32):
    for _i in range(16):
        _COLS_A[16 * _m + _i] = 32 * _m + _i
        _COLS_B[16 * _m + _i] = 32 * _m + 16 + _i


def _out_body(a0_ref, a1_ref, x_ref, attr_ref, wsc_ref, w2_ref, o_ref):
    o_ref[...] = (
        jnp.dot(a0_ref[...] + a1_ref[...], w2_ref[...],
                preferred_element_type=jnp.float32)
        + jnp.dot(x_ref[...], wsc_ref[...], preferred_element_type=jnp.float32)
        * attr_ref[...]
    )


_sc_mesh = plsc.VectorSubcoreMesh(core_axis_name="c", subcore_axis_name="s")


@functools.partial(
    pl.kernel,
    out_type=jax.ShapeDtypeStruct((_NC, _N, _D), jnp.float32),
    mesh=_sc_mesh,
    scratch_types=[
        pltpu.VMEM((_CPP, _C), jnp.int32),
        pltpu.VMEM((_CPP, _C), jnp.int32),
        pltpu.VMEM((2, _C, _D), jnp.float32),
        pltpu.VMEM((2, _C, _D), jnp.int32),
        pltpu.VMEM((2, _C, _D), jnp.float32),
        pltpu.VMEM((_ZC, _D), jnp.float32),
        pltpu.VMEM_SHARED((_N, _D), jnp.float32),
        pltpu.SemaphoreType.DMA,
        pltpu.SemaphoreType.DMA,
        pltpu.SemaphoreType.DMA,
        pltpu.SemaphoreType.DMA,
        pltpu.SemaphoreType.DMA,
        pltpu.SemaphoreType.DMA,
    ],
)
def _sc_gather_scatter(nf_hbm, wp_hbm, src_hbm, dst_hbm, out_hbm,
                       src_all, dst_all, rows_v, wp_v, prod_v, zero_v,
                       agg_sh, sg0, sg1, sw0, sw1, ss0, ss1):
    cid = lax.axis_index("c")
    sid = lax.axis_index("s")
    wid = cid * _NS + sid
    sg = (sg0, sg1)
    sw = (sw0, sw1)
    ss = (ss0, ss1)

    def _zfill(i, carry):
        for j in range(_D // 16):
            zero_v[i, pl.ds(j * 16, 16)] = jnp.zeros((16,), jnp.float32)
        return carry
    lax.fori_loop(0, _ZC, _zfill, 0)
    row0 = sid * _RPT

    def _zcopy(z, carry):
        pltpu.sync_copy(zero_v, agg_sh.at[pl.ds(row0 + z * _ZC, _ZC)])
        return carry
    lax.fori_loop(0, _RPT // _ZC, _zcopy, 0)

    @pl.when(sid == _NS - 1)
    def _zero_tail():
        pltpu.sync_copy(zero_v, agg_sh.at[pl.ds(_NS * _RPT, _ZC)])
        pltpu.sync_copy(zero_v, agg_sh.at[pl.ds(_NS * _RPT + _ZC, _ZC)])

    plsc.subcore_barrier()

    wp_base = sid * _EPW
    wp_cb = cid * (_D // 2)

    def _start_fetch(jg, j, b):
        pltpu.async_copy(nf_hbm.at[src_all.at[j]], rows_v.at[b], sg[b])
        pltpu.async_copy(
            wp_hbm.at[pl.ds(wp_base + jg * _C, _C)], wp_v.at[b], sw[b])

    def _finish(jg, j, b, first):
        pltpu.make_async_copy(nf_hbm.at[src_all.at[j]], rows_v.at[b], sg[b]).wait()
        pltpu.make_async_copy(
            wp_hbm.at[pl.ds(wp_base + jg * _C, _C)], wp_v.at[b], sw[b]).wait()
        if not first:
            pltpu.make_async_copy(
                prod_v.at[b], agg_sh.at[dst_all.at[j]], ss[b]).wait()

        rv = rows_v.at[b]
        wv = wp_v.at[b]
        pv = prod_v.at[b]

        def _mul(r, carry):
            for m in range(_D // 32):
                u = wv[r, pl.ds(wp_cb + m * 16, 16)]
                a = lax.bitcast_convert_type(u << 16, jnp.float32)
                bb = lax.bitcast_convert_type(u & jnp.int32(-65536), jnp.float32)
                sla = pl.ds(m * 32, 16)
                slb = pl.ds(m * 32 + 16, 16)
                pv[r, sla] = rv[r, sla] * a
                pv[r, slb] = rv[r, slb] * bb
            return carry
        lax.fori_loop(0, _C, _mul, 0)
        pltpu.async_copy(pv, agg_sh.at[dst_all.at[j]], ss[b], add=True)

    for p in range(_NPH):
        cbase = p * _CPP
        pltpu.sync_copy(src_hbm.at[wid, p], src_all)
        pltpu.sync_copy(dst_hbm.at[wid, p], dst_all)

        _start_fetch(cbase, 0, 0)
        _start_fetch(cbase + 1, 1, 1)
        if p == 0:
            _finish(cbase, 0, 0, True)
            _start_fetch(cbase + 2, 2, 0)
            _finish(cbase + 1, 1, 1, True)
            _start_fetch(cbase + 3, 3, 1)
            k0 = 1
        else:
            k0 = 0

        def _pair(k, carry):
            _finish(cbase + 2 * k, 2 * k, 0, False)

            @pl.when(k < _CPP // 2 - 1)
            def _pf0():
                _start_fetch(cbase + 2 * k + 2, 2 * k + 2, 0)

            _finish(cbase + 2 * k + 1, 2 * k + 1, 1, False)

            @pl.when(k < _CPP // 2 - 1)
            def _pf1():
                _start_fetch(cbase + 2 * k + 3, 2 * k + 3, 1)
            return carry

        lax.fori_loop(k0, _CPP // 2, _pair, 0)

    pltpu.make_async_copy(prod_v.at[0], agg_sh.at[dst_all.at[0]], ss[0]).wait()
    pltpu.make_async_copy(prod_v.at[1], agg_sh.at[dst_all.at[1]], ss[1]).wait()

    plsc.subcore_barrier()
    pltpu.sync_copy(agg_sh.at[pl.ds(row0, _RPT)],
                    out_hbm.at[cid, pl.ds(row0, _RPT)])

    @pl.when(sid == _NS - 1)
    def _copy_tail():
        pltpu.sync_copy(agg_sh.at[pl.ds(_NS * _RPT, 16)],
                        out_hbm.at[cid, pl.ds(_NS * _RPT, 16)])


def kernel(node_input, node_attr, edge_attr, edge_scalars, W1, Wfc1, Wfc2, W2, Wsc, edge_src, edge_dst):
    W1s = W1 * np.float32(1.0 / np.sqrt(_D))
    Wfc1s = Wfc1 * np.float32(1.0 / np.sqrt(_FC_IN))
    Wfc2s = Wfc2 * np.float32(1.0 / np.sqrt(_FC_HID))
    W2s = W2 * np.float32(1.0 / (np.sqrt(_D) * np.sqrt(_NUM_NEIGHBORS)))
    Wscs = Wsc * np.float32(1.0 / np.sqrt(_D))
    edge_src = edge_src.astype(jnp.int32).reshape(_NW, _NPH, _CPP, _C)
    edge_dst = edge_dst.astype(jnp.int32).reshape(_NW, _NPH, _CPP, _C)

    nf = pl.pallas_call(
        _node_body,
        grid=(5,),
        in_specs=[
            pl.BlockSpec((2000, _D), lambda i: (i, 0)),
            pl.BlockSpec((_D, _D), lambda i: (0, 0)),
        ],
        out_specs=pl.BlockSpec((2000, _D), lambda i: (i, 0)),
        out_shape=jax.ShapeDtypeStruct((_N, _D), jnp.float32),
    )(node_input, W1s)

    Wfc2a = Wfc2s[:, _COLS_A].astype(jnp.bfloat16)
    Wfc2b = Wfc2s[:, _COLS_B].astype(jnp.bfloat16)
    wp = pl.pallas_call(
        _fc_body,
        grid=(40,),
        in_specs=[
            pl.BlockSpec((4000, _FC_IN), lambda i: (i, 0)),
            pl.BlockSpec((4000, _FC_IN), lambda i: (40 + i, 0)),
            pl.BlockSpec((4000, 1), lambda i: (i, 0)),
            pl.BlockSpec((4000, 1), lambda i: (40 + i, 0)),
            pl.BlockSpec((_FC_IN, _FC_HID), lambda i: (0, 0)),
            pl.BlockSpec((_FC_HID, _D // 2), lambda i: (0, 0)),
            pl.BlockSpec((_FC_HID, _D // 2), lambda i: (0, 0)),
        ],
        out_specs=pl.BlockSpec((4000, _D), lambda i: (i, 0)),
        out_shape=jax.ShapeDtypeStruct((_E // 2, _D), jnp.int32),
    )(edge_scalars, edge_scalars, edge_attr, edge_attr,
      Wfc1s, Wfc2a, Wfc2b)

    agg = _sc_gather_scatter(nf, wp, edge_src, edge_dst)

    out = pl.pallas_call(
        _out_body,
        grid=(5,),
        in_specs=[
            pl.BlockSpec((2000, _D), lambda i: (i, 0)),
            pl.BlockSpec((2000, _D), lambda i: (i, 0)),
            pl.BlockSpec((2000, _D), lambda i: (i, 0)),
            pl.BlockSpec((2000, 1), lambda i: (i, 0)),
            pl.BlockSpec((_D, _D), lambda i: (0, 0)),
            pl.BlockSpec((_D, _D), lambda i: (0, 0)),
        ],
        out_specs=pl.BlockSpec((2000, _D), lambda i: (i, 0)),
        out_shape=jax.ShapeDtypeStruct((_N, _D), jnp.float32),
    )(agg[0], agg[1], node_input, node_attr, Wscs, W2s)
    return out


# speedup vs baseline: 1.0997x; 1.0997x over previous
import functools

import jax
import jax.numpy as jnp
import numpy as np
from jax import lax
from jax.experimental import pallas as pl
from jax.experimental.pallas import tpu as pltpu
from jax.experimental.pallas import tpu_sc as plsc

_N = 10000
_E = 320000
_D = 128
_FC_IN = 16
_FC_HID = 64
_NUM_NEIGHBORS = 32.0
_SILU_NORM = 1.679177

_NC = 2
_NS = 16
_NW = _NC * _NS
_EPW = _E // _NW
_C = 40
_NPH = 5
_CPP = 50
_RPT = 624
_ZC = 8


def _node_body(x_ref, w1_ref, nf_ref):
    nf_ref[...] = jnp.dot(x_ref[...], w1_ref[...], preferred_element_type=jnp.float32)


def _fc_body(es_ref, ea_ref, w1_ref, w2a_ref, w2b_ref, wp_ref):
    h = jnp.dot(es_ref[...], w1_ref[...], preferred_element_type=jnp.float32)
    h = (h * (1.0 / (1.0 + jnp.exp(-h)))) * _SILU_NORM
    hb = h.astype(jnp.bfloat16)
    ea = ea_ref[...]
    wpa = jnp.dot(hb, w2a_ref[...], preferred_element_type=jnp.float32) * ea
    wpb = jnp.dot(hb, w2b_ref[...], preferred_element_type=jnp.float32) * ea
    ia = lax.bitcast_convert_type(wpa.astype(jnp.bfloat16), jnp.uint16).astype(jnp.int32)
    ib = lax.bitcast_convert_type(wpb.astype(jnp.bfloat16), jnp.uint16).astype(jnp.int32)
    wp_ref[...] = ia | (ib << 16)


_COLS_A = np.empty(_D // 2, np.int32)
_COLS_B = np.empty(_D // 2, np.int32)
for _m in range(_D // 32):
    for _i in range(16):
        _COLS_A[16 * _m + _i] = 32 * _m + _i
        _COLS_B[16 * _m + _i] = 32 * _m + 16 + _i


def _out_body(a0_ref, a1_ref, x_ref, attr_ref, wsc_ref, w2_ref, o_ref):
    o_ref[...] = (
        jnp.dot(a0_ref[...] + a1_ref[...], w2_ref[...],
                preferred_element_type=jnp.float32)
        + jnp.dot(x_ref[...], wsc_ref[...], preferred_element_type=jnp.float32)
        * attr_ref[...]
    )


_sc_mesh = plsc.VectorSubcoreMesh(core_axis_name="c", subcore_axis_name="s")


@functools.partial(
    pl.kernel,
    out_type=jax.ShapeDtypeStruct((_NC, _N, _D), jnp.float32),
    mesh=_sc_mesh,
    scratch_types=[
        pltpu.VMEM((_CPP, _C), jnp.int32),
        pltpu.VMEM((_CPP, _C), jnp.int32),
        pltpu.VMEM((2, _C, _D), jnp.float32),
        pltpu.VMEM((2, _C, _D // 2), jnp.int32),
        pltpu.VMEM((2, _C, _D), jnp.float32),
        pltpu.VMEM((_ZC, _D), jnp.float32),
        pltpu.VMEM_SHARED((_N, _D), jnp.float32),
        pltpu.SemaphoreType.DMA,
        pltpu.SemaphoreType.DMA,
        pltpu.SemaphoreType.DMA,
        pltpu.SemaphoreType.DMA,
        pltpu.SemaphoreType.DMA,
        pltpu.SemaphoreType.DMA,
    ],
)
def _sc_gather_scatter(nf_hbm, wp_hbm, src_hbm, dst_hbm, out_hbm,
                       src_all, dst_all, rows_v, wp_v, prod_v, zero_v,
                       agg_sh, sg0, sg1, sw0, sw1, ss0, ss1):
    cid = lax.axis_index("c")
    sid = lax.axis_index("s")
    wid = cid * _NS + sid
    sg = (sg0, sg1)
    sw = (sw0, sw1)
    ss = (ss0, ss1)

    def _zfill(i, carry):
        for j in range(_D // 16):
            zero_v[i, pl.ds(j * 16, 16)] = jnp.zeros((16,), jnp.float32)
        return carry
    lax.fori_loop(0, _ZC, _zfill, 0)
    row0 = sid * _RPT

    def _zcopy(z, carry):
        pltpu.sync_copy(zero_v, agg_sh.at[pl.ds(row0 + z * _ZC, _ZC)])
        return carry
    lax.fori_loop(0, _RPT // _ZC, _zcopy, 0)

    @pl.when(sid == _NS - 1)
    def _zero_tail():
        pltpu.sync_copy(zero_v, agg_sh.at[pl.ds(_NS * _RPT, _ZC)])
        pltpu.sync_copy(zero_v, agg_sh.at[pl.ds(_NS * _RPT + _ZC, _ZC)])

    plsc.subcore_barrier()

    wp_base = wid * _EPW

    def _start_fetch(jg, j, b):
        pltpu.async_copy(nf_hbm.at[src_all.at[j]], rows_v.at[b], sg[b])
        pltpu.async_copy(
            wp_hbm.at[pl.ds(wp_base + jg * _C, _C)], wp_v.at[b], sw[b])

    def _finish(jg, j, b, first):
        pltpu.make_async_copy(nf_hbm.at[src_all.at[j]], rows_v.at[b], sg[b]).wait()
        pltpu.make_async_copy(
            wp_hbm.at[pl.ds(wp_base + jg * _C, _C)], wp_v.at[b], sw[b]).wait()
        if not first:
            pltpu.make_async_copy(
                prod_v.at[b], agg_sh.at[dst_all.at[j]], ss[b]).wait()

        rv = rows_v.at[b]
        wv = wp_v.at[b]
        pv = prod_v.at[b]

        def _mul(r2, carry):
            for dr in range(2):
                r = 2 * r2 + dr
                for m in range(_D // 32):
                    u = wv[r, pl.ds(m * 16, 16)]
                    a = lax.bitcast_convert_type(u << 16, jnp.float32)
                    bb = lax.bitcast_convert_type(u & jnp.int32(-65536), jnp.float32)
                    sla = pl.ds(m * 32, 16)
                    slb = pl.ds(m * 32 + 16, 16)
                    pv[r, sla] = rv[r, sla] * a
                    pv[r, slb] = rv[r, slb] * bb
            return carry
        lax.fori_loop(0, _C // 2, _mul, 0)
        pltpu.async_copy(pv, agg_sh.at[dst_all.at[j]], ss[b], add=True)

    for p in range(_NPH):
        cbase = p * _CPP
        pltpu.sync_copy(src_hbm.at[wid, p], src_all)
        pltpu.sync_copy(dst_hbm.at[wid, p], dst_all)

        _start_fetch(cbase, 0, 0)
        _start_fetch(cbase + 1, 1, 1)
        if p == 0:
            _finish(cbase, 0, 0, True)
            _start_fetch(cbase + 2, 2, 0)
            _finish(cbase + 1, 1, 1, True)
            _start_fetch(cbase + 3, 3, 1)
            k0 = 1
        else:
            k0 = 0

        def _pair(k, carry):
            _finish(cbase + 2 * k, 2 * k, 0, False)

            @pl.when(k < _CPP // 2 - 1)
            def _pf0():
                _start_fetch(cbase + 2 * k + 2, 2 * k + 2, 0)

            _finish(cbase + 2 * k + 1, 2 * k + 1, 1, False)

            @pl.when(k < _CPP // 2 - 1)
            def _pf1():
                _start_fetch(cbase + 2 * k + 3, 2 * k + 3, 1)
            return carry

        lax.fori_loop(k0, _CPP // 2, _pair, 0)

    pltpu.make_async_copy(prod_v.at[0], agg_sh.at[dst_all.at[0]], ss[0]).wait()
    pltpu.make_async_copy(prod_v.at[1], agg_sh.at[dst_all.at[1]], ss[1]).wait()

    plsc.subcore_barrier()
    pltpu.sync_copy(agg_sh.at[pl.ds(row0, _RPT)],
                    out_hbm.at[cid, pl.ds(row0, _RPT)])

    @pl.when(sid == _NS - 1)
    def _copy_tail():
        pltpu.sync_copy(agg_sh.at[pl.ds(_NS * _RPT, 16)],
                        out_hbm.at[cid, pl.ds(_NS * _RPT, 16)])


def kernel(node_input, node_attr, edge_attr, edge_scalars, W1, Wfc1, Wfc2, W2, Wsc, edge_src, edge_dst):
    W1s = W1 * np.float32(1.0 / np.sqrt(_D))
    Wfc1s = Wfc1 * np.float32(1.0 / np.sqrt(_FC_IN))
    Wfc2s = Wfc2 * np.float32(1.0 / np.sqrt(_FC_HID))
    W2s = W2 * np.float32(1.0 / (np.sqrt(_D) * np.sqrt(_NUM_NEIGHBORS)))
    Wscs = Wsc * np.float32(1.0 / np.sqrt(_D))
    edge_src = edge_src.astype(jnp.int32).reshape(_NW, _NPH, _CPP, _C)
    edge_dst = edge_dst.astype(jnp.int32).reshape(_NW, _NPH, _CPP, _C)

    nf = pl.pallas_call(
        _node_body,
        grid=(5,),
        in_specs=[
            pl.BlockSpec((2000, _D), lambda i: (i, 0)),
            pl.BlockSpec((_D, _D), lambda i: (0, 0)),
        ],
        out_specs=pl.BlockSpec((2000, _D), lambda i: (i, 0)),
        out_shape=jax.ShapeDtypeStruct((_N, _D), jnp.float32),
    )(node_input, W1s)

    Wfc2a = Wfc2s[:, _COLS_A].astype(jnp.bfloat16)
    Wfc2b = Wfc2s[:, _COLS_B].astype(jnp.bfloat16)
    wp = pl.pallas_call(
        _fc_body,
        grid=(40,),
        in_specs=[
            pl.BlockSpec((8000, _FC_IN), lambda i: (i, 0)),
            pl.BlockSpec((8000, 1), lambda i: (i, 0)),
            pl.BlockSpec((_FC_IN, _FC_HID), lambda i: (0, 0)),
            pl.BlockSpec((_FC_HID, _D // 2), lambda i: (0, 0)),
            pl.BlockSpec((_FC_HID, _D // 2), lambda i: (0, 0)),
        ],
        out_specs=pl.BlockSpec((8000, _D // 2), lambda i: (i, 0)),
        out_shape=jax.ShapeDtypeStruct((_E, _D // 2), jnp.int32),
    )(edge_scalars, edge_attr, Wfc1s, Wfc2a, Wfc2b)

    agg = _sc_gather_scatter(nf, wp, edge_src, edge_dst)

    out = pl.pallas_call(
        _out_body,
        grid=(5,),
        in_specs=[
            pl.BlockSpec((2000, _D), lambda i: (i, 0)),
            pl.BlockSpec((2000, _D), lambda i: (i, 0)),
            pl.BlockSpec((2000, _D), lambda i: (i, 0)),
            pl.BlockSpec((2000, 1), lambda i: (i, 0)),
            pl.BlockSpec((_D, _D), lambda i: (0, 0)),
            pl.BlockSpec((_D, _D), lambda i: (0, 0)),
        ],
        out_specs=pl.BlockSpec((2000, _D), lambda i: (i, 0)),
        out_shape=jax.ShapeDtypeStruct((_N, _D), jnp.float32),
    )(agg[0], agg[1], node_input, node_attr, Wscs, W2s)
    return out


# speedup vs baseline: 1.1244x; 1.0224x over previous
import functools

import jax
import jax.numpy as jnp
import numpy as np
from jax import lax
from jax.experimental import pallas as pl
from jax.experimental.pallas import tpu as pltpu
from jax.experimental.pallas import tpu_sc as plsc

_N = 10000
_E = 320000
_D = 128
_FC_IN = 16
_FC_HID = 64
_NUM_NEIGHBORS = 32.0
_SILU_NORM = 1.679177

_NC = 2
_NS = 16
_NW = _NC * _NS
_EPW = _E // _NW
_C = 40
_NPH = 5
_CPP = 50
_RPT = 624
_ZC = 8


def _node_body(x_ref, w1_ref, nf_ref):
    nf_ref[...] = jnp.dot(x_ref[...], w1_ref[...], preferred_element_type=jnp.float32)


def _fc_body(es_ref, ea_ref, w1_ref, w2_ref, wp_ref):
    h = jnp.dot(es_ref[...], w1_ref[...], preferred_element_type=jnp.float32)
    h = (h * (1.0 / (1.0 + jnp.exp(-h)))) * _SILU_NORM
    wp_ref[...] = jnp.dot(h, w2_ref[...], preferred_element_type=jnp.float32) * ea_ref[...]


def _out_body(a0_ref, a1_ref, x_ref, attr_ref, wsc_ref, w2_ref, o_ref):
    o_ref[...] = (
        jnp.dot(a0_ref[...] + a1_ref[...], w2_ref[...],
                preferred_element_type=jnp.float32)
        + jnp.dot(x_ref[...], wsc_ref[...], preferred_element_type=jnp.float32)
        * attr_ref[...]
    )


_sc_mesh = plsc.VectorSubcoreMesh(core_axis_name="c", subcore_axis_name="s")


@functools.partial(
    pl.kernel,
    out_type=jax.ShapeDtypeStruct((_NC, _N, _D), jnp.float32),
    mesh=_sc_mesh,
    scratch_types=[
        pltpu.VMEM((_CPP, _C), jnp.int32),
        pltpu.VMEM((_CPP, _C), jnp.int32),
        pltpu.VMEM((2, _C, _D), jnp.float32),
        pltpu.VMEM((2, _C, _D), jnp.float32),
        pltpu.VMEM((2, _C, _D), jnp.float32),
        pltpu.VMEM((_ZC, _D), jnp.float32),
        pltpu.VMEM_SHARED((_N, _D), jnp.float32),
        pltpu.SemaphoreType.DMA,
        pltpu.SemaphoreType.DMA,
        pltpu.SemaphoreType.DMA,
        pltpu.SemaphoreType.DMA,
        pltpu.SemaphoreType.DMA,
        pltpu.SemaphoreType.DMA,
    ],
)
def _sc_gather_scatter(nf_hbm, wp_hbm, src_hbm, dst_hbm, out_hbm,
                       src_all, dst_all, rows_v, wp_v, prod_v, zero_v,
                       agg_sh, sg0, sg1, sw0, sw1, ss0, ss1):
    cid = lax.axis_index("c")
    sid = lax.axis_index("s")
    wid = cid * _NS + sid
    sg = (sg0, sg1)
    sw = (sw0, sw1)
    ss = (ss0, ss1)

    def _zfill(i, carry):
        for j in range(_D // 16):
            zero_v[i, pl.ds(j * 16, 16)] = jnp.zeros((16,), jnp.float32)
        return carry
    lax.fori_loop(0, _ZC, _zfill, 0)
    row0 = sid * _RPT

    def _zcopy(z, carry):
        pltpu.sync_copy(zero_v, agg_sh.at[pl.ds(row0 + z * _ZC, _ZC)])
        return carry
    lax.fori_loop(0, _RPT // _ZC, _zcopy, 0)

    @pl.when(sid == _NS - 1)
    def _zero_tail():
        pltpu.sync_copy(zero_v, agg_sh.at[pl.ds(_NS * _RPT, _ZC)])
        pltpu.sync_copy(zero_v, agg_sh.at[pl.ds(_NS * _RPT + _ZC, _ZC)])

    plsc.subcore_barrier()

    wp_base = wid * _EPW

    def _start_fetch(jg, j, b):
        pltpu.async_copy(nf_hbm.at[src_all.at[j]], rows_v.at[b], sg[b])
        pltpu.async_copy(
            wp_hbm.at[pl.ds(wp_base + jg * _C, _C)], wp_v.at[b], sw[b])

    def _finish(jg, j, b, first):
        pltpu.make_async_copy(nf_hbm.at[src_all.at[j]], rows_v.at[b], sg[b]).wait()
        pltpu.make_async_copy(
            wp_hbm.at[pl.ds(wp_base + jg * _C, _C)], wp_v.at[b], sw[b]).wait()
        if not first:
            pltpu.make_async_copy(
                prod_v.at[b], agg_sh.at[dst_all.at[j]], ss[b]).wait()

        rv = rows_v.at[b]
        wv = wp_v.at[b]
        pv = prod_v.at[b]

        def _mul(r, carry):
            for jj in range(_D // 16):
                sl = pl.ds(jj * 16, 16)
                pv[r, sl] = rv[r, sl] * wv[r, sl]
            return carry
        lax.fori_loop(0, _C, _mul, 0)
        pltpu.async_copy(pv, agg_sh.at[dst_all.at[j]], ss[b], add=True)

    for p in range(_NPH):
        cbase = p * _CPP
        pltpu.sync_copy(src_hbm.at[wid, p], src_all)
        pltpu.sync_copy(dst_hbm.at[wid, p], dst_all)

        _start_fetch(cbase, 0, 0)
        _start_fetch(cbase + 1, 1, 1)
        if p == 0:
            _finish(cbase, 0, 0, True)
            _start_fetch(cbase + 2, 2, 0)
            _finish(cbase + 1, 1, 1, True)
            _start_fetch(cbase + 3, 3, 1)
            k0 = 1
        else:
            k0 = 0

        def _pair(k, carry):
            _finish(cbase + 2 * k, 2 * k, 0, False)

            @pl.when(k < _CPP // 2 - 1)
            def _pf0():
                _start_fetch(cbase + 2 * k + 2, 2 * k + 2, 0)

            _finish(cbase + 2 * k + 1, 2 * k + 1, 1, False)

            @pl.when(k < _CPP // 2 - 1)
            def _pf1():
                _start_fetch(cbase + 2 * k + 3, 2 * k + 3, 1)
            return carry

        lax.fori_loop(k0, _CPP // 2, _pair, 0)

    pltpu.make_async_copy(prod_v.at[0], agg_sh.at[dst_all.at[0]], ss[0]).wait()
    pltpu.make_async_copy(prod_v.at[1], agg_sh.at[dst_all.at[1]], ss[1]).wait()

    plsc.subcore_barrier()
    pltpu.sync_copy(agg_sh.at[pl.ds(row0, _RPT)],
                    out_hbm.at[cid, pl.ds(row0, _RPT)])

    @pl.when(sid == _NS - 1)
    def _copy_tail():
        pltpu.sync_copy(agg_sh.at[pl.ds(_NS * _RPT, 16)],
                        out_hbm.at[cid, pl.ds(_NS * _RPT, 16)])


def kernel(node_input, node_attr, edge_attr, edge_scalars, W1, Wfc1, Wfc2, W2, Wsc, edge_src, edge_dst):
    W1s = W1 * np.float32(1.0 / np.sqrt(_D))
    Wfc1s = Wfc1 * np.float32(1.0 / np.sqrt(_FC_IN))
    Wfc2s = Wfc2 * np.float32(1.0 / np.sqrt(_FC_HID))
    W2s = W2 * np.float32(1.0 / (np.sqrt(_D) * np.sqrt(_NUM_NEIGHBORS)))
    Wscs = Wsc * np.float32(1.0 / np.sqrt(_D))
    edge_src = edge_src.astype(jnp.int32).reshape(_NW, _NPH, _CPP, _C)
    edge_dst = edge_dst.astype(jnp.int32).reshape(_NW, _NPH, _CPP, _C)

    nf = pl.pallas_call(
        _node_body,
        grid=(5,),
        in_specs=[
            pl.BlockSpec((2000, _D), lambda i: (i, 0)),
            pl.BlockSpec((_D, _D), lambda i: (0, 0)),
        ],
        out_specs=pl.BlockSpec((2000, _D), lambda i: (i, 0)),
        out_shape=jax.ShapeDtypeStruct((_N, _D), jnp.float32),
    )(node_input, W1s)

    wp = pl.pallas_call(
        _fc_body,
        grid=(40,),
        in_specs=[
            pl.BlockSpec((8000, _FC_IN), lambda i: (i, 0)),
            pl.BlockSpec((8000, 1), lambda i: (i, 0)),
            pl.BlockSpec((_FC_IN, _FC_HID), lambda i: (0, 0)),
            pl.BlockSpec((_FC_HID, _D), lambda i: (0, 0)),
        ],
        out_specs=pl.BlockSpec((8000, _D), lambda i: (i, 0)),
        out_shape=jax.ShapeDtypeStruct((_E, _D), jnp.float32),
    )(edge_scalars, edge_attr, Wfc1s, Wfc2s)

    agg = _sc_gather_scatter(nf, wp, edge_src, edge_dst)

    out = pl.pallas_call(
        _out_body,
        grid=(5,),
        in_specs=[
            pl.BlockSpec((2000, _D), lambda i: (i, 0)),
            pl.BlockSpec((2000, _D), lambda i: (i, 0)),
            pl.BlockSpec((2000, _D), lambda i: (i, 0)),
            pl.BlockSpec((2000, 1), lambda i: (i, 0)),
            pl.BlockSpec((_D, _D), lambda i: (0, 0)),
            pl.BlockSpec((_D, _D), lambda i: (0, 0)),
        ],
        out_specs=pl.BlockSpec((2000, _D), lambda i: (i, 0)),
        out_shape=jax.ShapeDtypeStruct((_N, _D), jnp.float32),
    )(agg[0], agg[1], node_input, node_attr, Wscs, W2s)
    return out
